# Initial kernel scaffold; baseline (speedup 1.0000x reference)
import jax
import jax.numpy as jnp
from jax.experimental import pallas as pl


def kernel(x, edge_index, edge_attr, Wz, bz, Wlz, blz, Wr, br, Wlr, blr, Wh, bh, Wlh, blh, W1, b1, W2, b2):
    raise NotImplementedError("write your pallas kernel here")


# speedup vs baseline: 15.9991x; 15.9991x over previous
import functools

import jax
import jax.numpy as jnp
from jax import lax
from jax.experimental import pallas as pl
from jax.experimental.pallas import tpu as pltpu
from jax.experimental.pallas import tpu_sc as plsc

N = 10000
E = 320000
F = 128
HID = 128
DE = 16
NCLS = 4

NC = 2
NS = 16
NPAD = 10240
RPT = NPAD // NS
EPC = E // NC
EPW = E // (NC * NS)
KH = 2000
KS = 200
KE = 400

_MESH = plsc.VectorSubcoreMesh(
    core_axis_name="c", subcore_axis_name="s", num_cores=NC, num_subcores=NS)


@functools.partial(
    pl.kernel,
    out_type=jax.ShapeDtypeStruct((NC, NPAD), jnp.float32),
    mesh=_MESH,
    scratch_types=[
        pltpu.VMEM((KH,), jnp.int32),
        pltpu.VMEM((KH,), jnp.float32),
        pltpu.VMEM((RPT,), jnp.float32),
        pltpu.VMEM_SHARED((NPAD,), jnp.float32),
    ],
)
def _sc_degree(dst_hbm, out_hbm, idx_v, ones_v, zero_v, deg_sh):
    c = lax.axis_index("c")
    s = lax.axis_index("s")

    def zfill(i, carry):
        zero_v[pl.ds(i * 16, 16)] = jnp.zeros((16,), jnp.float32)
        return carry

    lax.fori_loop(0, RPT // 16, zfill, None)

    def ofill(i, carry):
        ones_v[pl.ds(i * 16, 16)] = jnp.ones((16,), jnp.float32)
        return carry

    lax.fori_loop(0, KH // 16, ofill, None)

    pltpu.sync_copy(zero_v, deg_sh.at[pl.ds(s * RPT, RPT)])
    plsc.subcore_barrier()

    base = c * EPC + s * EPW

    def step(i, carry):
        pltpu.sync_copy(dst_hbm.at[pl.ds(base + i * KH, KH)], idx_v)
        pltpu.sync_copy(ones_v, deg_sh.at[idx_v], add=True)
        return carry

    lax.fori_loop(0, EPW // KH, step, None)

    plsc.subcore_barrier()
    pltpu.sync_copy(deg_sh.at[pl.ds(s * RPT, RPT)],
                    out_hbm.at[c, pl.ds(s * RPT, RPT)])


@functools.partial(
    pl.kernel,
    out_type=jax.ShapeDtypeStruct((NC, NPAD, F), jnp.float32),
    mesh=_MESH,
    scratch_types=[
        pltpu.VMEM((KS,), jnp.int32),
        pltpu.VMEM((KS,), jnp.int32),
        pltpu.VMEM((KS, F), jnp.float32),
        pltpu.VMEM_SHARED((NPAD, F), jnp.float32),
        pltpu.SemaphoreType.DMA,
    ],
)
def _sc_scatter(y_hbm, src_hbm, dst_hbm, out_hbm, isrc_v, idst_v, rows_v,
                acc_sh, sem):
    c = lax.axis_index("c")
    s = lax.axis_index("s")

    def zfill(i, carry):
        r = i // (F // 16)
        k = i % (F // 16)
        rows_v[r, pl.ds(k * 16, 16)] = jnp.zeros((16,), jnp.float32)
        return carry

    lax.fori_loop(0, KS * (F // 16), zfill, None)
    for off, sz in ((0, KS), (KS, KS), (2 * KS, KS), (3 * KS, RPT - 3 * KS)):
        pltpu.sync_copy(rows_v.at[pl.ds(0, sz)],
                        acc_sh.at[pl.ds(s * RPT + off, sz)])
    plsc.subcore_barrier()

    base = c * EPC + s * EPW

    def step(i, carry):
        e0 = base + i * KS
        pltpu.sync_copy(src_hbm.at[pl.ds(e0, KS)], isrc_v)
        pltpu.sync_copy(dst_hbm.at[pl.ds(e0, KS)], idst_v)
        pltpu.async_copy(y_hbm.at[isrc_v], rows_v, sem).wait()
        pltpu.sync_copy(rows_v, acc_sh.at[idst_v], add=True)
        return carry

    lax.fori_loop(0, EPW // KS, step, None)

    plsc.subcore_barrier()
    pltpu.sync_copy(acc_sh.at[pl.ds(s * RPT, RPT)],
                    out_hbm.at[c, pl.ds(s * RPT, RPT)])


@functools.partial(
    pl.kernel,
    out_type=jax.ShapeDtypeStruct((E, F), jnp.float32),
    mesh=_MESH,
    scratch_types=[
        pltpu.VMEM((KE,), jnp.int32),
        pltpu.VMEM((KE,), jnp.int32),
        pltpu.VMEM((KE, F), jnp.float32),
        pltpu.SemaphoreType.DMA,
    ],
)
def _sc_edge(p_hbm, q_hbm, src_hbm, dst_hbm, out_hbm, isrc_v, idst_v, buf_v,
             sem):
    c = lax.axis_index("c")
    s = lax.axis_index("s")
    base = c * EPC + s * EPW

    def step(i, carry):
        e0 = base + i * KE
        pltpu.sync_copy(src_hbm.at[pl.ds(e0, KE)], isrc_v)
        pltpu.sync_copy(dst_hbm.at[pl.ds(e0, KE)], idst_v)
        pltpu.async_copy(p_hbm.at[isrc_v], buf_v, sem).wait()
        pltpu.async_copy(q_hbm.at[idst_v], buf_v, sem, add=True).wait()
        pltpu.sync_copy(buf_v, out_hbm.at[pl.ds(e0, KE)])
        return carry

    lax.fori_loop(0, EPW // KE, step, None)


def _prescale_body(x_ref, da_ref, db_ref, y_ref, dinv_ref):
    dinv = lax.rsqrt(da_ref[...] + db_ref[...] + 1.0)
    y_ref[...] = x_ref[...] * dinv
    dinv_ref[...] = dinv


def _tc_prescale(x, dega, degb):
    B = 1000
    return pl.pallas_call(
        _prescale_body,
        grid=(N // B,),
        in_specs=[
            pl.BlockSpec((B, F), lambda i: (i, 0)),
            pl.BlockSpec((B, 1), lambda i: (i, 0)),
            pl.BlockSpec((B, 1), lambda i: (i, 0)),
        ],
        out_specs=[
            pl.BlockSpec((B, F), lambda i: (i, 0)),
            pl.BlockSpec((B, 1), lambda i: (i, 0)),
        ],
        out_shape=[
            jax.ShapeDtypeStruct((N, F), jnp.float32),
            jax.ShapeDtypeStruct((N, 1), jnp.float32),
        ],
    )(x, dega, degb)


def _dense_body(sa_ref, sb_ref, x_ref, dinv_ref, wz_ref, bz_ref,
                wlz_ref, blz_ref, wh_ref, bh_ref, wlh_ref, blh_ref,
                w1s_ref, w1d_ref, p_ref, q_ref):
    dinv = dinv_ref[...]
    xa = dinv * (sa_ref[0] + sb_ref[0]) + (dinv * dinv) * x_ref[...]
    cz = jnp.dot(xa, wz_ref[...], preferred_element_type=jnp.float32)
    z = jax.nn.sigmoid(
        jnp.dot(cz + bz_ref[...], wlz_ref[...],
                preferred_element_type=jnp.float32) + blz_ref[...])
    ch = jnp.dot(xa, wh_ref[...], preferred_element_type=jnp.float32)
    ht = jnp.tanh(
        jnp.dot(ch + bh_ref[...], wlh_ref[...],
                preferred_element_type=jnp.float32) + blh_ref[...])
    h = (1.0 - z) * ht
    p_ref[...] = jnp.dot(h, w1s_ref[...], preferred_element_type=jnp.float32)
    q_ref[...] = jnp.dot(h, w1d_ref[...], preferred_element_type=jnp.float32)


def _tc_dense(s2, x, dinv, wz, bz, wlz, blz, wh, bh, wlh, blh, w1s, w1d):
    B = 1000
    row = lambda i: (i, 0)
    full = pl.BlockSpec((HID, HID), lambda i: (0, 0))
    bias = pl.BlockSpec((1, HID), lambda i: (0, 0))
    return pl.pallas_call(
        _dense_body,
        grid=(N // B,),
        in_specs=[
            pl.BlockSpec((1, B, F), lambda i: (0, i, 0)),
            pl.BlockSpec((1, B, F), lambda i: (1, i, 0)),
            pl.BlockSpec((B, F), row),
            pl.BlockSpec((B, 1), row),
            full, bias, full, bias, full, bias, full, bias, full, full,
        ],
        out_specs=[
            pl.BlockSpec((B, HID), row),
            pl.BlockSpec((B, HID), row),
        ],
        out_shape=[
            jax.ShapeDtypeStruct((N, HID), jnp.float32),
            jax.ShapeDtypeStruct((N, HID), jnp.float32),
        ],
    )(s2, s2, x, dinv, wz, bz, wlz, blz, wh, bh, wlh, blh, w1s, w1d)


def _emlp_body(t_ref, ea_ref, w1e_ref, b1_ref, w2_ref, b2_ref, o_ref):
    hid = jnp.maximum(
        t_ref[...]
        + jnp.dot(ea_ref[...], w1e_ref[...],
                  preferred_element_type=jnp.float32)
        + b1_ref[...], 0.0)
    o_ref[...] = jnp.dot(hid, w2_ref[...],
                         preferred_element_type=jnp.float32) + b2_ref[...]


def _tc_edge_mlp(t0, ea, w1e, b1, w2, b2):
    B = 2000
    return pl.pallas_call(
        _emlp_body,
        grid=(E // B,),
        in_specs=[
            pl.BlockSpec((B, F), lambda i: (i, 0)),
            pl.BlockSpec((B, DE), lambda i: (i, 0)),
            pl.BlockSpec((DE, HID), lambda i: (0, 0)),
            pl.BlockSpec((1, HID), lambda i: (0, 0)),
            pl.BlockSpec((HID, NCLS), lambda i: (0, 0)),
            pl.BlockSpec((1, NCLS), lambda i: (0, 0)),
        ],
        out_specs=pl.BlockSpec((B, NCLS), lambda i: (i, 0)),
        out_shape=jax.ShapeDtypeStruct((E, NCLS), jnp.float32),
    )(t0, ea, w1e, b1, w2, b2)


def kernel(x, edge_index, edge_attr, Wz, bz, Wlz, blz, Wr, br, Wlr, blr,
           Wh, bh, Wlh, blh, W1, b1, W2, b2):
    src = edge_index[0]
    dst = edge_index[1]

    deg2 = _sc_degree(dst)
    dega = deg2[0].reshape(NPAD, 1)
    degb = deg2[1].reshape(NPAD, 1)

    y, dinv = _tc_prescale(x, dega, degb)

    s2 = _sc_scatter(y, src, dst)

    p, q = _tc_dense(
        s2, x, dinv,
        Wz, bz.reshape(1, HID), Wlz[:HID], blz.reshape(1, HID),
        Wh, bh.reshape(1, HID), Wlh[:HID], blh.reshape(1, HID),
        W1[:HID], W1[HID:2 * HID])

    t0 = _sc_edge(p, q, src, dst)

    return _tc_edge_mlp(t0, edge_attr, W1[2 * HID:], b1.reshape(1, HID),
                        W2, b2.reshape(1, NCLS))


# speedup vs baseline: 17.4655x; 1.0917x over previous
import functools

import jax
import jax.numpy as jnp
from jax import lax
from jax.experimental import pallas as pl
from jax.experimental.pallas import tpu as pltpu
from jax.experimental.pallas import tpu_sc as plsc

N = 10000
E = 320000
F = 128
HID = 128
DE = 16
NCLS = 4

NC = 2
NS = 16
NPAD = 10240
RPT = NPAD // NS
EPC = E // NC
EPW = E // (NC * NS)
KH = 2000
KS = 80
NCHS = EPW // KS
KE = 200
NCHE = EPW // KE

_MESH = plsc.VectorSubcoreMesh(
    core_axis_name="c", subcore_axis_name="s", num_cores=NC, num_subcores=NS)


@functools.partial(
    pl.kernel,
    out_type=jax.ShapeDtypeStruct((NC, NPAD), jnp.float32),
    mesh=_MESH,
    scratch_types=[
        pltpu.VMEM((KH,), jnp.int32),
        pltpu.VMEM((KH,), jnp.float32),
        pltpu.VMEM((RPT,), jnp.float32),
        pltpu.VMEM_SHARED((NPAD,), jnp.float32),
    ],
)
def _sc_degree(dst_hbm, out_hbm, idx_v, ones_v, zero_v, deg_sh):
    c = lax.axis_index("c")
    s = lax.axis_index("s")

    def zfill(i, carry):
        zero_v[pl.ds(i * 16, 16)] = jnp.zeros((16,), jnp.float32)
        return carry

    lax.fori_loop(0, RPT // 16, zfill, None)

    def ofill(i, carry):
        ones_v[pl.ds(i * 16, 16)] = jnp.ones((16,), jnp.float32)
        return carry

    lax.fori_loop(0, KH // 16, ofill, None)

    pltpu.sync_copy(zero_v, deg_sh.at[pl.ds(s * RPT, RPT)])
    plsc.subcore_barrier()

    base = c * EPC + s * EPW

    def step(i, carry):
        pltpu.sync_copy(dst_hbm.at[pl.ds(base + i * KH, KH)], idx_v)
        pltpu.sync_copy(ones_v, deg_sh.at[idx_v], add=True)
        return carry

    lax.fori_loop(0, EPW // KH, step, None)

    plsc.subcore_barrier()
    pltpu.sync_copy(deg_sh.at[pl.ds(s * RPT, RPT)],
                    out_hbm.at[c, pl.ds(s * RPT, RPT)])


@functools.partial(
    pl.kernel,
    out_type=jax.ShapeDtypeStruct((NC, NPAD, F), jnp.float32),
    mesh=_MESH,
    scratch_types=[
        pltpu.VMEM((KS,), jnp.int32),
        pltpu.VMEM((KS,), jnp.int32),
        pltpu.VMEM((KS,), jnp.int32),
        pltpu.VMEM((KS,), jnp.int32),
        pltpu.VMEM((2, KS, F), jnp.float32),
        pltpu.VMEM_SHARED((NPAD, F), jnp.float32),
        pltpu.SemaphoreType.DMA,
        pltpu.SemaphoreType.DMA,
        pltpu.SemaphoreType.DMA,
        pltpu.SemaphoreType.DMA,
    ],
)
def _sc_scatter(y_hbm, src_hbm, dst_hbm, out_hbm, isrc0, isrc1, idst0, idst1,
                rows_v, acc_sh, is0, is1, gs0, gs1):
    c = lax.axis_index("c")
    s = lax.axis_index("s")

    def zfill(i, carry):
        r = i // (F // 16)
        k = i % (F // 16)
        rows_v[0, r, pl.ds(k * 16, 16)] = jnp.zeros((16,), jnp.float32)
        return carry

    lax.fori_loop(0, KS * (F // 16), zfill, None)
    for j in range(RPT // KS):
        pltpu.sync_copy(rows_v.at[0],
                        acc_sh.at[pl.ds(s * RPT + j * KS, KS)])
    plsc.subcore_barrier()

    base = c * EPC + s * EPW
    isrc = (isrc0, isrc1)
    idst = (idst0, idst1)
    isem = (is0, is1)
    gsem = (gs0, gs1)

    def stage_a(chunk, b):
        e0 = base + chunk * KS
        pltpu.async_copy(src_hbm.at[pl.ds(e0, KS)], isrc[b], isem[b])
        pltpu.async_copy(dst_hbm.at[pl.ds(e0, KS)], idst[b], isem[b])

    def stage_b(chunk, b):
        e0 = base + chunk * KS
        pltpu.make_async_copy(src_hbm.at[pl.ds(e0, KS)], isrc[b],
                              isem[b]).wait()
        pltpu.make_async_copy(dst_hbm.at[pl.ds(e0, KS)], idst[b],
                              isem[b]).wait()
        pltpu.async_copy(y_hbm.at[isrc[b]], rows_v.at[b], gsem[b])

    def stage_c(chunk, b):
        pltpu.make_async_copy(y_hbm.at[isrc[b]], rows_v.at[b],
                              gsem[b]).wait()
        pltpu.sync_copy(rows_v.at[b], acc_sh.at[idst[b]], add=True)

    stage_a(0, 0)
    stage_a(1, 1)
    stage_b(0, 0)

    def step(j, carry):
        ca = 1 + 2 * j
        stage_b(ca, 1)
        stage_c(ca - 1, 0)
        stage_a(ca + 1, 0)
        stage_b(ca + 1, 0)
        stage_c(ca, 1)
        stage_a(ca + 2, 1)
        return carry

    lax.fori_loop(0, (NCHS - 3) // 2, step, None)
    stage_b(NCHS - 2, 1)
    stage_c(NCHS - 3, 0)
    stage_a(NCHS - 1, 0)
    stage_b(NCHS - 1, 0)
    stage_c(NCHS - 2, 1)
    stage_c(NCHS - 1, 0)

    plsc.subcore_barrier()
    pltpu.sync_copy(acc_sh.at[pl.ds(s * RPT, RPT)],
                    out_hbm.at[c, pl.ds(s * RPT, RPT)])


@functools.partial(
    pl.kernel,
    out_type=jax.ShapeDtypeStruct((E, F), jnp.float32),
    mesh=_MESH,
    scratch_types=[
        pltpu.VMEM((EPW,), jnp.int32),
        pltpu.VMEM((EPW,), jnp.int32),
        pltpu.VMEM((2, KE, F), jnp.float32),
        pltpu.SemaphoreType.DMA,
        pltpu.SemaphoreType.DMA,
        pltpu.SemaphoreType.DMA,
        pltpu.SemaphoreType.DMA,
    ],
)
def _sc_edge(p_hbm, q_hbm, src_hbm, dst_hbm, out_hbm, isrc_v, idst_v, buf_v,
             gs0, gs1, ss0, ss1):
    c = lax.axis_index("c")
    s = lax.axis_index("s")
    base = c * EPC + s * EPW

    pltpu.sync_copy(src_hbm.at[pl.ds(base, EPW)], isrc_v)
    pltpu.sync_copy(dst_hbm.at[pl.ds(base, EPW)], idst_v)

    gsems = (gs0, gs1)
    ssems = (ss0, ss1)

    def _compute(i, b):
        o = i * KE
        pltpu.async_copy(p_hbm.at[isrc_v.at[pl.ds(o, KE)]], buf_v.at[b],
                         gsems[b]).wait()
        pltpu.async_copy(q_hbm.at[idst_v.at[pl.ds(o, KE)]], buf_v.at[b],
                         gsems[b], add=True).wait()

    def _store(i, b):
        pltpu.async_copy(buf_v.at[b], out_hbm.at[pl.ds(base + i * KE, KE)],
                         ssems[b])

    def _store_wait(b):
        pltpu.make_async_copy(buf_v.at[b], out_hbm.at[pl.ds(base, KE)],
                              ssems[b]).wait()

    _compute(0, 0)
    _store(0, 0)
    _compute(1, 1)
    _store(1, 1)

    def step(j, carry):
        ca = 2 + 2 * j
        _store_wait(0)
        _compute(ca, 0)
        _store(ca, 0)
        _store_wait(1)
        _compute(ca + 1, 1)
        _store(ca + 1, 1)
        return carry

    lax.fori_loop(0, (NCHE - 2) // 2, step, None)
    _store_wait(0)
    _store_wait(1)


def _prescale_body(x_ref, da_ref, db_ref, y_ref, dinv_ref):
    dinv = lax.rsqrt(da_ref[...] + db_ref[...] + 1.0)
    y_ref[...] = x_ref[...] * dinv
    dinv_ref[...] = dinv


def _tc_prescale(x, dega, degb):
    B = 1000
    return pl.pallas_call(
        _prescale_body,
        grid=(N // B,),
        in_specs=[
            pl.BlockSpec((B, F), lambda i: (i, 0)),
            pl.BlockSpec((B, 1), lambda i: (i, 0)),
            pl.BlockSpec((B, 1), lambda i: (i, 0)),
        ],
        out_specs=[
            pl.BlockSpec((B, F), lambda i: (i, 0)),
            pl.BlockSpec((B, 1), lambda i: (i, 0)),
        ],
        out_shape=[
            jax.ShapeDtypeStruct((N, F), jnp.float32),
            jax.ShapeDtypeStruct((N, 1), jnp.float32),
        ],
    )(x, dega, degb)


def _dense_body(sa_ref, sb_ref, x_ref, dinv_ref, wz_ref, bz_ref,
                wlz_ref, blz_ref, wh_ref, bh_ref, wlh_ref, blh_ref,
                w1s_ref, w1d_ref, p_ref, q_ref):
    dinv = dinv_ref[...]
    xa = dinv * (sa_ref[0] + sb_ref[0]) + (dinv * dinv) * x_ref[...]
    cz = jnp.dot(xa, wz_ref[...], preferred_element_type=jnp.float32)
    z = jax.nn.sigmoid(
        jnp.dot(cz + bz_ref[...], wlz_ref[...],
                preferred_element_type=jnp.float32) + blz_ref[...])
    ch = jnp.dot(xa, wh_ref[...], preferred_element_type=jnp.float32)
    ht = jnp.tanh(
        jnp.dot(ch + bh_ref[...], wlh_ref[...],
                preferred_element_type=jnp.float32) + blh_ref[...])
    h = (1.0 - z) * ht
    p_ref[...] = jnp.dot(h, w1s_ref[...], preferred_element_type=jnp.float32)
    q_ref[...] = jnp.dot(h, w1d_ref[...], preferred_element_type=jnp.float32)


def _tc_dense(s2, x, dinv, wz, bz, wlz, blz, wh, bh, wlh, blh, w1s, w1d):
    B = 1000
    row = lambda i: (i, 0)
    full = pl.BlockSpec((HID, HID), lambda i: (0, 0))
    bias = pl.BlockSpec((1, HID), lambda i: (0, 0))
    return pl.pallas_call(
        _dense_body,
        grid=(N // B,),
        in_specs=[
            pl.BlockSpec((1, B, F), lambda i: (0, i, 0)),
            pl.BlockSpec((1, B, F), lambda i: (1, i, 0)),
            pl.BlockSpec((B, F), row),
            pl.BlockSpec((B, 1), row),
            full, bias, full, bias, full, bias, full, bias, full, full,
        ],
        out_specs=[
            pl.BlockSpec((B, HID), row),
            pl.BlockSpec((B, HID), row),
        ],
        out_shape=[
            jax.ShapeDtypeStruct((N, HID), jnp.float32),
            jax.ShapeDtypeStruct((N, HID), jnp.float32),
        ],
    )(s2, s2, x, dinv, wz, bz, wlz, blz, wh, bh, wlh, blh, w1s, w1d)


def _emlp_body(t_ref, ea_ref, w1e_ref, b1_ref, w2_ref, b2_ref, o_ref):
    hid = jnp.maximum(
        t_ref[...]
        + jnp.dot(ea_ref[...], w1e_ref[...],
                  preferred_element_type=jnp.float32)
        + b1_ref[...], 0.0)
    o_ref[...] = jnp.dot(hid, w2_ref[...],
                         preferred_element_type=jnp.float32) + b2_ref[...]


def _tc_edge_mlp(t0, ea, w1e, b1, w2, b2):
    B = 2000
    return pl.pallas_call(
        _emlp_body,
        grid=(E // B,),
        in_specs=[
            pl.BlockSpec((B, F), lambda i: (i, 0)),
            pl.BlockSpec((B, DE), lambda i: (i, 0)),
            pl.BlockSpec((DE, HID), lambda i: (0, 0)),
            pl.BlockSpec((1, HID), lambda i: (0, 0)),
            pl.BlockSpec((HID, NCLS), lambda i: (0, 0)),
            pl.BlockSpec((1, NCLS), lambda i: (0, 0)),
        ],
        out_specs=pl.BlockSpec((B, NCLS), lambda i: (i, 0)),
        out_shape=jax.ShapeDtypeStruct((E, NCLS), jnp.float32),
    )(t0, ea, w1e, b1, w2, b2)


def kernel(x, edge_index, edge_attr, Wz, bz, Wlz, blz, Wr, br, Wlr, blr,
           Wh, bh, Wlh, blh, W1, b1, W2, b2):
    src = edge_index[0]
    dst = edge_index[1]

    deg2 = _sc_degree(dst)
    dega = deg2[0].reshape(NPAD, 1)
    degb = deg2[1].reshape(NPAD, 1)

    y, dinv = _tc_prescale(x, dega, degb)

    s2 = _sc_scatter(y, src, dst)

    p, q = _tc_dense(
        s2, x, dinv,
        Wz, bz.reshape(1, HID), Wlz[:HID], blz.reshape(1, HID),
        Wh, bh.reshape(1, HID), Wlh[:HID], blh.reshape(1, HID),
        W1[:HID], W1[HID:2 * HID])

    t0 = _sc_edge(p, q, src, dst)

    return _tc_edge_mlp(t0, edge_attr, W1[2 * HID:], b1.reshape(1, HID),
                        W2, b2.reshape(1, NCLS))


# speedup vs baseline: 17.8393x; 1.0214x over previous
import functools

import jax
import jax.numpy as jnp
from jax import lax
from jax.experimental import pallas as pl
from jax.experimental.pallas import tpu as pltpu
from jax.experimental.pallas import tpu_sc as plsc

N = 10000
E = 320000
F = 128
HID = 128
DE = 16
NCLS = 4

NC = 2
NS = 16
NPAD = 10240
RPT = NPAD // NS
EPC = E // NC
EPW = E // (NC * NS)
KH = 2000
KS = 80
NCHS = EPW // KS
KE = 200
NCHE = EPW // KE

_MESH = plsc.VectorSubcoreMesh(
    core_axis_name="c", subcore_axis_name="s", num_cores=NC, num_subcores=NS)


@functools.partial(
    pl.kernel,
    out_type=jax.ShapeDtypeStruct((NC, NPAD), jnp.float32),
    mesh=_MESH,
    scratch_types=[
        pltpu.VMEM((KH,), jnp.int32),
        pltpu.VMEM((KH,), jnp.float32),
        pltpu.VMEM((RPT,), jnp.float32),
        pltpu.VMEM_SHARED((NPAD,), jnp.float32),
    ],
)
def _sc_degree(dst_hbm, out_hbm, idx_v, ones_v, zero_v, deg_sh):
    c = lax.axis_index("c")
    s = lax.axis_index("s")

    def zfill(i, carry):
        zero_v[pl.ds(i * 16, 16)] = jnp.zeros((16,), jnp.float32)
        return carry

    lax.fori_loop(0, RPT // 16, zfill, None)

    def ofill(i, carry):
        ones_v[pl.ds(i * 16, 16)] = jnp.ones((16,), jnp.float32)
        return carry

    lax.fori_loop(0, KH // 16, ofill, None)

    pltpu.sync_copy(zero_v, deg_sh.at[pl.ds(s * RPT, RPT)])
    plsc.subcore_barrier()

    base = c * EPC + s * EPW

    def step(i, carry):
        pltpu.sync_copy(dst_hbm.at[pl.ds(base + i * KH, KH)], idx_v)
        pltpu.sync_copy(ones_v, deg_sh.at[idx_v], add=True)
        return carry

    lax.fori_loop(0, EPW // KH, step, None)

    plsc.subcore_barrier()
    pltpu.sync_copy(deg_sh.at[pl.ds(s * RPT, RPT)],
                    out_hbm.at[c, pl.ds(s * RPT, RPT)])


@functools.partial(
    pl.kernel,
    out_type=jax.ShapeDtypeStruct((NC, NPAD, F), jnp.float32),
    mesh=_MESH,
    scratch_types=[
        pltpu.VMEM((KS,), jnp.int32),
        pltpu.VMEM((KS,), jnp.int32),
        pltpu.VMEM((KS,), jnp.int32),
        pltpu.VMEM((KS,), jnp.int32),
        pltpu.VMEM((2, KS, F), jnp.float32),
        pltpu.VMEM_SHARED((NPAD, F), jnp.float32),
        pltpu.SemaphoreType.DMA,
        pltpu.SemaphoreType.DMA,
        pltpu.SemaphoreType.DMA,
        pltpu.SemaphoreType.DMA,
    ],
)
def _sc_scatter(y_hbm, src_hbm, dst_hbm, out_hbm, isrc0, isrc1, idst0, idst1,
                rows_v, acc_sh, is0, is1, gs0, gs1):
    c = lax.axis_index("c")
    s = lax.axis_index("s")

    def zfill(i, carry):
        r = i // (F // 16)
        k = i % (F // 16)
        rows_v[0, r, pl.ds(k * 16, 16)] = jnp.zeros((16,), jnp.float32)
        return carry

    lax.fori_loop(0, KS * (F // 16), zfill, None)
    for j in range(RPT // KS):
        pltpu.sync_copy(rows_v.at[0],
                        acc_sh.at[pl.ds(s * RPT + j * KS, KS)])
    plsc.subcore_barrier()

    base = c * EPC + s * EPW
    isrc = (isrc0, isrc1)
    idst = (idst0, idst1)
    isem = (is0, is1)
    gsem = (gs0, gs1)

    def stage_a(chunk, b):
        e0 = base + chunk * KS
        pltpu.async_copy(src_hbm.at[pl.ds(e0, KS)], isrc[b], isem[b])
        pltpu.async_copy(dst_hbm.at[pl.ds(e0, KS)], idst[b], isem[b])

    def stage_b(chunk, b):
        e0 = base + chunk * KS
        pltpu.make_async_copy(src_hbm.at[pl.ds(e0, KS)], isrc[b],
                              isem[b]).wait()
        pltpu.make_async_copy(dst_hbm.at[pl.ds(e0, KS)], idst[b],
                              isem[b]).wait()
        pltpu.async_copy(y_hbm.at[isrc[b]], rows_v.at[b], gsem[b])

    def stage_c(chunk, b):
        pltpu.make_async_copy(y_hbm.at[isrc[b]], rows_v.at[b],
                              gsem[b]).wait()
        pltpu.sync_copy(rows_v.at[b], acc_sh.at[idst[b]], add=True)

    stage_a(0, 0)
    stage_a(1, 1)
    stage_b(0, 0)

    def step(j, carry):
        ca = 1 + 2 * j
        stage_b(ca, 1)
        stage_c(ca - 1, 0)
        stage_a(ca + 1, 0)
        stage_b(ca + 1, 0)
        stage_c(ca, 1)
        stage_a(ca + 2, 1)
        return carry

    lax.fori_loop(0, (NCHS - 3) // 2, step, None)
    stage_b(NCHS - 2, 1)
    stage_c(NCHS - 3, 0)
    stage_a(NCHS - 1, 0)
    stage_b(NCHS - 1, 0)
    stage_c(NCHS - 2, 1)
    stage_c(NCHS - 1, 0)

    plsc.subcore_barrier()
    pltpu.sync_copy(acc_sh.at[pl.ds(s * RPT, RPT)],
                    out_hbm.at[c, pl.ds(s * RPT, RPT)])


@functools.partial(
    pl.kernel,
    out_type=jax.ShapeDtypeStruct((E, F), jnp.float32),
    mesh=_MESH,
    scratch_types=[
        pltpu.VMEM((EPW,), jnp.int32),
        pltpu.VMEM((EPW,), jnp.int32),
        pltpu.VMEM((2, KE, F), jnp.float32),
        pltpu.SemaphoreType.DMA,
        pltpu.SemaphoreType.DMA,
        pltpu.SemaphoreType.DMA,
        pltpu.SemaphoreType.DMA,
    ],
)
def _sc_edge(p_hbm, q_hbm, src_hbm, dst_hbm, out_hbm, isrc_v, idst_v, buf_v,
             gs0, gs1, ss0, ss1):
    c = lax.axis_index("c")
    s = lax.axis_index("s")
    base = c * EPC + s * EPW

    pltpu.sync_copy(src_hbm.at[pl.ds(base, EPW)], isrc_v)
    pltpu.sync_copy(dst_hbm.at[pl.ds(base, EPW)], idst_v)

    gsems = (gs0, gs1)
    ssems = (ss0, ss1)

    def _p_start(i, b):
        pltpu.async_copy(p_hbm.at[isrc_v.at[pl.ds(i * KE, KE)]],
                         buf_v.at[b], gsems[b])

    def _p_wait(i, b):
        pltpu.make_async_copy(p_hbm.at[isrc_v.at[pl.ds(i * KE, KE)]],
                              buf_v.at[b], gsems[b]).wait()

    def _q_start(i, b):
        pltpu.async_copy(q_hbm.at[idst_v.at[pl.ds(i * KE, KE)]],
                         buf_v.at[b], gsems[b], add=True)

    def _store(i, b):
        pltpu.async_copy(buf_v.at[b], out_hbm.at[pl.ds(base + i * KE, KE)],
                         ssems[b])

    def _store_wait(b):
        pltpu.make_async_copy(buf_v.at[b], out_hbm.at[pl.ds(base, KE)],
                              ssems[b]).wait()

    _p_start(0, 0)
    _store(1, 1)

    def step(j, carry):
        c = 2 * j
        _p_wait(c, 0)
        _q_start(c, 0)
        _store_wait(1)
        _p_start(c + 1, 1)
        _p_wait(c, 0)
        _store(c, 0)
        _p_wait(c + 1, 1)
        _q_start(c + 1, 1)
        _store_wait(0)
        _p_start(c + 2, 0)
        _p_wait(c + 1, 1)
        _store(c + 1, 1)
        return carry

    lax.fori_loop(0, (NCHE - 2) // 2, step, None)
    c = NCHE - 2
    _p_wait(c, 0)
    _q_start(c, 0)
    _store_wait(1)
    _p_start(c + 1, 1)
    _p_wait(c, 0)
    _store(c, 0)
    _p_wait(c + 1, 1)
    _q_start(c + 1, 1)
    _p_wait(c + 1, 1)
    _store(c + 1, 1)
    _store_wait(0)
    _store_wait(1)


def _prescale_body(x_ref, da_ref, db_ref, y_ref, dinv_ref):
    dinv = lax.rsqrt(da_ref[...] + db_ref[...] + 1.0)
    y_ref[...] = x_ref[...] * dinv
    dinv_ref[...] = dinv


def _tc_prescale(x, dega, degb):
    B = 1000
    return pl.pallas_call(
        _prescale_body,
        grid=(N // B,),
        in_specs=[
            pl.BlockSpec((B, F), lambda i: (i, 0)),
            pl.BlockSpec((B, 1), lambda i: (i, 0)),
            pl.BlockSpec((B, 1), lambda i: (i, 0)),
        ],
        out_specs=[
            pl.BlockSpec((B, F), lambda i: (i, 0)),
            pl.BlockSpec((B, 1), lambda i: (i, 0)),
        ],
        out_shape=[
            jax.ShapeDtypeStruct((N, F), jnp.float32),
            jax.ShapeDtypeStruct((N, 1), jnp.float32),
        ],
    )(x, dega, degb)


def _dense_body(sa_ref, sb_ref, x_ref, dinv_ref, wz_ref, bz_ref,
                wlz_ref, blz_ref, wh_ref, bh_ref, wlh_ref, blh_ref,
                w1s_ref, w1d_ref, p_ref, q_ref):
    dinv = dinv_ref[...]
    xa = dinv * (sa_ref[0] + sb_ref[0]) + (dinv * dinv) * x_ref[...]
    cz = jnp.dot(xa, wz_ref[...], preferred_element_type=jnp.float32)
    z = jax.nn.sigmoid(
        jnp.dot(cz + bz_ref[...], wlz_ref[...],
                preferred_element_type=jnp.float32) + blz_ref[...])
    ch = jnp.dot(xa, wh_ref[...], preferred_element_type=jnp.float32)
    ht = jnp.tanh(
        jnp.dot(ch + bh_ref[...], wlh_ref[...],
                preferred_element_type=jnp.float32) + blh_ref[...])
    h = (1.0 - z) * ht
    p_ref[...] = jnp.dot(h, w1s_ref[...], preferred_element_type=jnp.float32)
    q_ref[...] = jnp.dot(h, w1d_ref[...], preferred_element_type=jnp.float32)


def _tc_dense(s2, x, dinv, wz, bz, wlz, blz, wh, bh, wlh, blh, w1s, w1d):
    B = 1000
    row = lambda i: (i, 0)
    full = pl.BlockSpec((HID, HID), lambda i: (0, 0))
    bias = pl.BlockSpec((1, HID), lambda i: (0, 0))
    return pl.pallas_call(
        _dense_body,
        grid=(N // B,),
        in_specs=[
            pl.BlockSpec((1, B, F), lambda i: (0, i, 0)),
            pl.BlockSpec((1, B, F), lambda i: (1, i, 0)),
            pl.BlockSpec((B, F), row),
            pl.BlockSpec((B, 1), row),
            full, bias, full, bias, full, bias, full, bias, full, full,
        ],
        out_specs=[
            pl.BlockSpec((B, HID), row),
            pl.BlockSpec((B, HID), row),
        ],
        out_shape=[
            jax.ShapeDtypeStruct((N, HID), jnp.float32),
            jax.ShapeDtypeStruct((N, HID), jnp.float32),
        ],
    )(s2, s2, x, dinv, wz, bz, wlz, blz, wh, bh, wlh, blh, w1s, w1d)


def _emlp_body(t_ref, ea_ref, w1e_ref, b1_ref, w2_ref, b2_ref, o_ref):
    hid = jnp.maximum(
        t_ref[...]
        + jnp.dot(ea_ref[...], w1e_ref[...],
                  preferred_element_type=jnp.float32)
        + b1_ref[...], 0.0)
    o_ref[...] = jnp.dot(hid, w2_ref[...],
                         preferred_element_type=jnp.float32) + b2_ref[...]


def _tc_edge_mlp(t0, ea, w1e, b1, w2, b2):
    B = 2000
    return pl.pallas_call(
        _emlp_body,
        grid=(E // B,),
        in_specs=[
            pl.BlockSpec((B, F), lambda i: (i, 0)),
            pl.BlockSpec((B, DE), lambda i: (i, 0)),
            pl.BlockSpec((DE, HID), lambda i: (0, 0)),
            pl.BlockSpec((1, HID), lambda i: (0, 0)),
            pl.BlockSpec((HID, NCLS), lambda i: (0, 0)),
            pl.BlockSpec((1, NCLS), lambda i: (0, 0)),
        ],
        out_specs=pl.BlockSpec((B, NCLS), lambda i: (i, 0)),
        out_shape=jax.ShapeDtypeStruct((E, NCLS), jnp.float32),
    )(t0, ea, w1e, b1, w2, b2)


def kernel(x, edge_index, edge_attr, Wz, bz, Wlz, blz, Wr, br, Wlr, blr,
           Wh, bh, Wlh, blh, W1, b1, W2, b2):
    src = edge_index[0]
    dst = edge_index[1]

    deg2 = _sc_degree(dst)
    dega = deg2[0].reshape(NPAD, 1)
    degb = deg2[1].reshape(NPAD, 1)

    y, dinv = _tc_prescale(x, dega, degb)

    s2 = _sc_scatter(y, src, dst)

    p, q = _tc_dense(
        s2, x, dinv,
        Wz, bz.reshape(1, HID), Wlz[:HID], blz.reshape(1, HID),
        Wh, bh.reshape(1, HID), Wlh[:HID], blh.reshape(1, HID),
        W1[:HID], W1[HID:2 * HID])

    t0 = _sc_edge(p, q, src, dst)

    return _tc_edge_mlp(t0, edge_attr, W1[2 * HID:], b1.reshape(1, HID),
                        W2, b2.reshape(1, NCLS))


# speedup vs baseline: 18.8056x; 1.0542x over previous
import functools

import jax
import jax.numpy as jnp
from jax import lax
from jax.experimental import pallas as pl
from jax.experimental.pallas import tpu as pltpu
from jax.experimental.pallas import tpu_sc as plsc

N = 10000
E = 320000
F = 128
HID = 128
DE = 16
NCLS = 4

NC = 2
NS = 16
NPAD = 10240
RPT = NPAD // NS
EPC = E // NC
EPW = E // (NC * NS)
KH = 2000
KS = 80
NCHS = EPW // KS
KE = 200
NCHE = EPW // KE

_MESH = plsc.VectorSubcoreMesh(
    core_axis_name="c", subcore_axis_name="s", num_cores=NC, num_subcores=NS)


@functools.partial(
    pl.kernel,
    out_type=jax.ShapeDtypeStruct((NC, NPAD), jnp.float32),
    mesh=_MESH,
    scratch_types=[
        pltpu.VMEM((KH,), jnp.int32),
        pltpu.VMEM((KH,), jnp.float32),
        pltpu.VMEM((RPT,), jnp.float32),
        pltpu.VMEM_SHARED((NPAD,), jnp.float32),
    ],
)
def _sc_degree(dst_hbm, out_hbm, idx_v, ones_v, zero_v, deg_sh):
    c = lax.axis_index("c")
    s = lax.axis_index("s")

    def zfill(i, carry):
        zero_v[pl.ds(i * 16, 16)] = jnp.zeros((16,), jnp.float32)
        return carry

    lax.fori_loop(0, RPT // 16, zfill, None)

    def ofill(i, carry):
        ones_v[pl.ds(i * 16, 16)] = jnp.ones((16,), jnp.float32)
        return carry

    lax.fori_loop(0, KH // 16, ofill, None)

    pltpu.sync_copy(zero_v, deg_sh.at[pl.ds(s * RPT, RPT)])
    plsc.subcore_barrier()

    base = c * EPC + s * EPW

    def step(i, carry):
        pltpu.sync_copy(dst_hbm.at[pl.ds(base + i * KH, KH)], idx_v)
        pltpu.sync_copy(ones_v, deg_sh.at[idx_v], add=True)
        return carry

    lax.fori_loop(0, EPW // KH, step, None)

    plsc.subcore_barrier()
    pltpu.sync_copy(deg_sh.at[pl.ds(s * RPT, RPT)],
                    out_hbm.at[c, pl.ds(s * RPT, RPT)])


@functools.partial(
    pl.kernel,
    out_type=jax.ShapeDtypeStruct((NC, NPAD, F), jnp.float32),
    mesh=_MESH,
    scratch_types=[
        pltpu.VMEM((KS,), jnp.int32),
        pltpu.VMEM((KS,), jnp.int32),
        pltpu.VMEM((KS,), jnp.int32),
        pltpu.VMEM((KS,), jnp.int32),
        pltpu.VMEM((2, KS, F), jnp.float32),
        pltpu.VMEM_SHARED((NPAD, F), jnp.float32),
        pltpu.SemaphoreType.DMA,
        pltpu.SemaphoreType.DMA,
        pltpu.SemaphoreType.DMA,
        pltpu.SemaphoreType.DMA,
    ],
)
def _sc_scatter(y_hbm, src_hbm, dst_hbm, out_hbm, isrc0, isrc1, idst0, idst1,
                rows_v, acc_sh, is0, is1, gs0, gs1):
    c = lax.axis_index("c")
    s = lax.axis_index("s")

    def zfill(i, carry):
        r = i // (F // 16)
        k = i % (F // 16)
        rows_v[0, r, pl.ds(k * 16, 16)] = jnp.zeros((16,), jnp.float32)
        return carry

    lax.fori_loop(0, KS * (F // 16), zfill, None)
    for j in range(RPT // KS):
        pltpu.sync_copy(rows_v.at[0],
                        acc_sh.at[pl.ds(s * RPT + j * KS, KS)])
    plsc.subcore_barrier()

    base = c * EPC + s * EPW
    isrc = (isrc0, isrc1)
    idst = (idst0, idst1)
    isem = (is0, is1)
    gsem = (gs0, gs1)

    def stage_a(chunk, b):
        e0 = base + chunk * KS
        pltpu.async_copy(src_hbm.at[pl.ds(e0, KS)], isrc[b], isem[b])
        pltpu.async_copy(dst_hbm.at[pl.ds(e0, KS)], idst[b], isem[b])

    def stage_b(chunk, b):
        e0 = base + chunk * KS
        pltpu.make_async_copy(src_hbm.at[pl.ds(e0, KS)], isrc[b],
                              isem[b]).wait()
        pltpu.make_async_copy(dst_hbm.at[pl.ds(e0, KS)], idst[b],
                              isem[b]).wait()
        pltpu.async_copy(y_hbm.at[isrc[b]], rows_v.at[b], gsem[b])

    def stage_c(chunk, b):
        pltpu.make_async_copy(y_hbm.at[isrc[b]], rows_v.at[b],
                              gsem[b]).wait()
        pltpu.sync_copy(rows_v.at[b], acc_sh.at[idst[b]], add=True)

    stage_a(0, 0)
    stage_a(1, 1)
    stage_b(0, 0)

    def step(j, carry):
        ca = 1 + 2 * j
        stage_b(ca, 1)
        stage_c(ca - 1, 0)
        stage_a(ca + 1, 0)
        stage_b(ca + 1, 0)
        stage_c(ca, 1)
        stage_a(ca + 2, 1)
        return carry

    lax.fori_loop(0, (NCHS - 3) // 2, step, None)
    stage_b(NCHS - 2, 1)
    stage_c(NCHS - 3, 0)
    stage_a(NCHS - 1, 0)
    stage_b(NCHS - 1, 0)
    stage_c(NCHS - 2, 1)
    stage_c(NCHS - 1, 0)

    plsc.subcore_barrier()
    pltpu.sync_copy(acc_sh.at[pl.ds(s * RPT, RPT)],
                    out_hbm.at[c, pl.ds(s * RPT, RPT)])


def _make_sc_edge(e_base, e_count):
    epc = e_count // NC
    epw = e_count // (NC * NS)
    nche = epw // KE

    @functools.partial(
        pl.kernel,
        out_type=jax.ShapeDtypeStruct((e_count, F), jnp.float32),
        mesh=_MESH,
        scratch_types=[
            pltpu.VMEM((epw,), jnp.int32),
            pltpu.VMEM((epw,), jnp.int32),
            pltpu.VMEM((2, KE, F), jnp.float32),
            pltpu.SemaphoreType.DMA,
            pltpu.SemaphoreType.DMA,
            pltpu.SemaphoreType.DMA,
            pltpu.SemaphoreType.DMA,
        ],
    )
    def _sc_edge(p_hbm, q_hbm, src_hbm, dst_hbm, out_hbm, isrc_v, idst_v,
                 buf_v, gs0, gs1, ss0, ss1):
        c = lax.axis_index("c")
        s = lax.axis_index("s")
        gbase = e_base + c * epc + s * epw
        obase = c * epc + s * epw

        pltpu.sync_copy(src_hbm.at[pl.ds(gbase, epw)], isrc_v)
        pltpu.sync_copy(dst_hbm.at[pl.ds(gbase, epw)], idst_v)

        gsems = (gs0, gs1)
        ssems = (ss0, ss1)

        def _p_start(i, b):
            pltpu.async_copy(p_hbm.at[isrc_v.at[pl.ds(i * KE, KE)]],
                             buf_v.at[b], gsems[b])

        def _p_wait(i, b):
            pltpu.make_async_copy(p_hbm.at[isrc_v.at[pl.ds(i * KE, KE)]],
                                  buf_v.at[b], gsems[b]).wait()

        def _q_start(i, b):
            pltpu.async_copy(q_hbm.at[idst_v.at[pl.ds(i * KE, KE)]],
                             buf_v.at[b], gsems[b], add=True)

        def _store(i, b):
            pltpu.async_copy(buf_v.at[b],
                             out_hbm.at[pl.ds(obase + i * KE, KE)], ssems[b])

        def _store_wait(b):
            pltpu.make_async_copy(buf_v.at[b], out_hbm.at[pl.ds(obase, KE)],
                                  ssems[b]).wait()

        def _chunk_even(cc):
            _p_wait(cc, 0)
            _q_start(cc, 0)
            _store_wait(1)
            _p_start(cc + 1, 1)
            _p_wait(cc, 0)
            _store(cc, 0)

        def _chunk_odd(cc, start_next):
            _p_wait(cc, 1)
            _q_start(cc, 1)
            _store_wait(0)
            if start_next:
                _p_start(cc + 1, 0)
            _p_wait(cc, 1)
            _store(cc, 1)

        _p_start(0, 0)
        _store(1, 1)

        def step(j, carry):
            cc = 2 * j
            _chunk_even(cc)
            _chunk_odd(cc + 1, True)
            return carry

        if nche % 2 == 0:
            lax.fori_loop(0, (nche - 2) // 2, step, None)
            _chunk_even(nche - 2)
            _p_wait(nche - 1, 1)
            _q_start(nche - 1, 1)
            _p_wait(nche - 1, 1)
            _store(nche - 1, 1)
            _store_wait(0)
            _store_wait(1)
        else:
            lax.fori_loop(0, (nche - 3) // 2, step, None)
            _chunk_even(nche - 3)
            _chunk_odd(nche - 2, True)
            _p_wait(nche - 1, 0)
            _q_start(nche - 1, 0)
            _p_wait(nche - 1, 0)
            _store(nche - 1, 0)
            _store_wait(1)
            _store_wait(0)

    return _sc_edge


_sc_edge_half0 = _make_sc_edge(0, E // 2)
_sc_edge_half1 = _make_sc_edge(E // 2, E // 2)


def _prescale_body(x_ref, da_ref, db_ref, y_ref, dinv_ref):
    dinv = lax.rsqrt(da_ref[...] + db_ref[...] + 1.0)
    y_ref[...] = x_ref[...] * dinv
    dinv_ref[...] = dinv


def _tc_prescale(x, dega, degb):
    B = 1000
    return pl.pallas_call(
        _prescale_body,
        grid=(N // B,),
        in_specs=[
            pl.BlockSpec((B, F), lambda i: (i, 0)),
            pl.BlockSpec((B, 1), lambda i: (i, 0)),
            pl.BlockSpec((B, 1), lambda i: (i, 0)),
        ],
        out_specs=[
            pl.BlockSpec((B, F), lambda i: (i, 0)),
            pl.BlockSpec((B, 1), lambda i: (i, 0)),
        ],
        out_shape=[
            jax.ShapeDtypeStruct((N, F), jnp.float32),
            jax.ShapeDtypeStruct((N, 1), jnp.float32),
        ],
    )(x, dega, degb)


def _dense_body(sa_ref, sb_ref, x_ref, dinv_ref, wz_ref, bz_ref,
                wlz_ref, blz_ref, wh_ref, bh_ref, wlh_ref, blh_ref,
                w1s_ref, w1d_ref, p_ref, q_ref):
    dinv = dinv_ref[...]
    xa = dinv * (sa_ref[0] + sb_ref[0]) + (dinv * dinv) * x_ref[...]
    cz = jnp.dot(xa, wz_ref[...], preferred_element_type=jnp.float32)
    z = jax.nn.sigmoid(
        jnp.dot(cz + bz_ref[...], wlz_ref[...],
                preferred_element_type=jnp.float32) + blz_ref[...])
    ch = jnp.dot(xa, wh_ref[...], preferred_element_type=jnp.float32)
    ht = jnp.tanh(
        jnp.dot(ch + bh_ref[...], wlh_ref[...],
                preferred_element_type=jnp.float32) + blh_ref[...])
    h = (1.0 - z) * ht
    p_ref[...] = jnp.dot(h, w1s_ref[...], preferred_element_type=jnp.float32)
    q_ref[...] = jnp.dot(h, w1d_ref[...], preferred_element_type=jnp.float32)


def _tc_dense(s2, x, dinv, wz, bz, wlz, blz, wh, bh, wlh, blh, w1s, w1d):
    B = 1000
    row = lambda i: (i, 0)
    full = pl.BlockSpec((HID, HID), lambda i: (0, 0))
    bias = pl.BlockSpec((1, HID), lambda i: (0, 0))
    return pl.pallas_call(
        _dense_body,
        grid=(N // B,),
        in_specs=[
            pl.BlockSpec((1, B, F), lambda i: (0, i, 0)),
            pl.BlockSpec((1, B, F), lambda i: (1, i, 0)),
            pl.BlockSpec((B, F), row),
            pl.BlockSpec((B, 1), row),
            full, bias, full, bias, full, bias, full, bias, full, full,
        ],
        out_specs=[
            pl.BlockSpec((B, HID), row),
            pl.BlockSpec((B, HID), row),
        ],
        out_shape=[
            jax.ShapeDtypeStruct((N, HID), jnp.float32),
            jax.ShapeDtypeStruct((N, HID), jnp.float32),
        ],
    )(s2, s2, x, dinv, wz, bz, wlz, blz, wh, bh, wlh, blh, w1s, w1d)


def _emlp_body(t_ref, ea_ref, w1e_ref, b1_ref, w2_ref, b2_ref, o_ref):
    hid = jnp.maximum(
        t_ref[...]
        + jnp.dot(ea_ref[...], w1e_ref[...],
                  preferred_element_type=jnp.float32)
        + b1_ref[...], 0.0)
    o_ref[...] = jnp.dot(hid, w2_ref[...],
                         preferred_element_type=jnp.float32) + b2_ref[...]


def _tc_edge_mlp(t0, ea, w1e, b1, w2, b2):
    ne = t0.shape[0]
    B = 8000
    return pl.pallas_call(
        _emlp_body,
        grid=(ne // B,),
        in_specs=[
            pl.BlockSpec((B, F), lambda i: (i, 0)),
            pl.BlockSpec((B, DE), lambda i: (i, 0)),
            pl.BlockSpec((DE, HID), lambda i: (0, 0)),
            pl.BlockSpec((1, HID), lambda i: (0, 0)),
            pl.BlockSpec((HID, NCLS), lambda i: (0, 0)),
            pl.BlockSpec((1, NCLS), lambda i: (0, 0)),
        ],
        out_specs=pl.BlockSpec((B, NCLS), lambda i: (i, 0)),
        out_shape=jax.ShapeDtypeStruct((ne, NCLS), jnp.float32),
    )(t0, ea, w1e, b1, w2, b2)


def kernel(x, edge_index, edge_attr, Wz, bz, Wlz, blz, Wr, br, Wlr, blr,
           Wh, bh, Wlh, blh, W1, b1, W2, b2):
    src = edge_index[0]
    dst = edge_index[1]

    deg2 = _sc_degree(dst)
    dega = deg2[0].reshape(NPAD, 1)
    degb = deg2[1].reshape(NPAD, 1)

    y, dinv = _tc_prescale(x, dega, degb)

    s2 = _sc_scatter(y, src, dst)

    p, q = _tc_dense(
        s2, x, dinv,
        Wz, bz.reshape(1, HID), Wlz[:HID], blz.reshape(1, HID),
        Wh, bh.reshape(1, HID), Wlh[:HID], blh.reshape(1, HID),
        W1[:HID], W1[HID:2 * HID])

    w1e = W1[2 * HID:]
    b1r = b1.reshape(1, HID)
    b2r = b2.reshape(1, NCLS)
    eh = E // 2
    t0a = _sc_edge_half0(p, q, src, dst)
    t0b = _sc_edge_half1(p, q, src, dst)
    outa = _tc_edge_mlp(t0a, edge_attr[:eh], w1e, b1r, W2, b2r)
    outb = _tc_edge_mlp(t0b, edge_attr[eh:], w1e, b1r, W2, b2r)
    return jnp.concatenate([outa, outb], axis=0)
